# baseline (speedup 1.0000x reference)
import jax
import jax.numpy as jnp
from jax import lax
from jax.experimental import pallas as pl
from jax.experimental.pallas import tpu as pltpu
from jax.experimental.pallas import tpu_sc as plsc

_BATCH = 16384
_FIELDS = 26
_CHUNK_OUT = 32
_N_CHUNKS = 4
_OUT_DIM = _N_CHUNKS * _CHUNK_OUT
_B_FLAT = _BATCH * _FIELDS
_L = 128
_NW = 32
_ROWS_PER_W = _B_FLAT // (_NW * _L)
_TROWS = 1000000
_BR = 2048

_mesh = plsc.VectorSubcoreMesh(core_axis_name="core", subcore_axis_name="subcore")


def _concat_tables(t0, t1, t2, t3):
    return jnp.concatenate([t0, t1, t2, t3], axis=1)


@jax.jit
def kernel(indices, table_0, table_1, table_2, table_3):
    idx = indices.reshape(_B_FLAT // _L, _L).astype(jnp.int32)
    tcat = _concat_tables(table_0, table_1, table_2, table_3)

    @pl.kernel(
        out_type=jax.ShapeDtypeStruct((_B_FLAT, _OUT_DIM), jnp.float32),
        mesh=_mesh,
        scratch_types=[
            pltpu.VMEM((2, _L), jnp.int32),
            pltpu.VMEM((_L, _OUT_DIM), jnp.float32),
            pltpu.VMEM((_L, _OUT_DIM), jnp.float32),
            pltpu.SemaphoreType.DMA,
            pltpu.SemaphoreType.DMA,
            pltpu.SemaphoreType.DMA,
            pltpu.SemaphoreType.DMA,
            pltpu.SemaphoreType.DMA,
        ],
    )
    def k(idx_hbm, t_hbm, o_hbm, idx_v, gb0, gb1, sg0, sg1, si, so0, so1):
        gbs = (gb0, gb1)
        sgs = (sg0, sg1)
        sos = (so0, so1)
        wid = lax.axis_index("subcore") * 2 + lax.axis_index("core")
        row0 = wid * _ROWS_PER_W

        def chunk_body(c, p):
            pn = 1 - p
            irow = row0 + c

            @pl.when(c < _ROWS_PER_W - 1)
            def _():
                pltpu.async_copy(idx_hbm.at[irow + 1], idx_v.at[pn], si)

            pltpu.make_async_copy(t_hbm.at[idx_v.at[p]], gbs[p], sgs[p]).wait()
            pltpu.async_copy(gbs[p], o_hbm.at[pl.ds(irow * _L, _L)], sos[p])

            @pl.when(c < _ROWS_PER_W - 1)
            def _():
                pltpu.make_async_copy(idx_hbm.at[irow + 1], idx_v.at[pn], si).wait()

                @pl.when(c >= 1)
                def _():
                    pltpu.make_async_copy(
                        gbs[pn], o_hbm.at[pl.ds((irow - 1) * _L, _L)], sos[pn]
                    ).wait()

                pltpu.async_copy(t_hbm.at[idx_v.at[pn]], gbs[pn], sgs[pn])

        pltpu.sync_copy(idx_hbm.at[row0], idx_v.at[0])
        pltpu.async_copy(t_hbm.at[idx_v.at[0]], gb0, sg0)

        @pl.loop(0, _ROWS_PER_W // 2)
        def _(cc):
            chunk_body(cc * 2, 0)
            chunk_body(cc * 2 + 1, 1)

        last = row0 + _ROWS_PER_W - 1
        pltpu.make_async_copy(
            gb0, o_hbm.at[pl.ds((last - 1) * _L, _L)], so0
        ).wait()
        pltpu.make_async_copy(
            gb1, o_hbm.at[pl.ds(last * _L, _L)], so1
        ).wait()

    out = k(idx, tcat)
    return out.reshape(_BATCH, _FIELDS, _OUT_DIM)
